# baseline (speedup 1.0000x reference)
import functools

import jax
import jax.numpy as jnp
from jax.experimental import pallas as pl
from jax.experimental.pallas import tpu as pltpu

N = 10000
E = 320000
B = 100
D = 128
CHUNK = 16000
NCHUNK = E // CHUNK

_CP = pltpu.CompilerParams(
    dimension_semantics=("arbitrary",),
    vmem_limit_bytes=128 * 1024 * 1024,
)


def _scatter_loop(e_ref, acc_ref, src_ref, combine):
    def body(e, carry):
        s = e_ref[0, e]
        d = e_ref[1, e]
        acc_ref[pl.ds(d, 1), :] = combine(
            acc_ref[pl.ds(d, 1), :], src_ref[pl.ds(s, 1), :]
        )
        return carry

    jax.lax.fori_loop(0, CHUNK, body, 0)


def _bn(t, gamma, beta):
    mu = jnp.mean(t, axis=0, keepdims=True)
    var = jnp.var(t, axis=0, keepdims=True)
    return gamma * (t - mu) / jnp.sqrt(var + 1e-3) + beta


def _conv1_kernel(x_ref, wn_ref, ws_ref, b_ref, g_ref, be_ref, e_ref,
                  out_ref, xw_ref, acc_ref):
    i = pl.program_id(0)

    @pl.when(i == 0)
    def _():
        xw_ref[...] = jnp.dot(x_ref[...], wn_ref[...],
                              preferred_element_type=jnp.float32)
        acc_ref[...] = jnp.zeros_like(acc_ref)

    _scatter_loop(e_ref, acc_ref, xw_ref, lambda a, b: a + b)

    @pl.when(i == NCHUNK - 1)
    def _():
        t = jnp.dot(x_ref[...], ws_ref[...],
                    preferred_element_type=jnp.float32)
        t = jnp.maximum(t + acc_ref[...] + b_ref[...], 0.0)
        out_ref[...] = _bn(t, g_ref[...], be_ref[...])


def _pool_kernel(h_ref, e_ref, out_ref, acc_ref):
    i = pl.program_id(0)

    @pl.when(i == 0)
    def _():
        acc_ref[...] = jnp.full(acc_ref.shape, -jnp.inf, dtype=jnp.float32)

    _scatter_loop(e_ref, acc_ref, h_ref, jnp.maximum)

    @pl.when(i == NCHUNK - 1)
    def _():
        out_ref[...] = jnp.maximum(h_ref[...], acc_ref[...])


def _conv2_kernel(p_ref, wsr_ref, wnr_ref, b_ref, g_ref, be_ref, e_ref,
                  out_ref, xw_ref, acc_ref):
    i = pl.program_id(0)

    @pl.when(i == 0)
    def _():
        cn = jnp.sum(p_ref[...] * wnr_ref[...], axis=1, keepdims=True)
        xw_ref[...] = jnp.broadcast_to(cn, (N, D))
        acc_ref[...] = jnp.zeros_like(acc_ref)

    _scatter_loop(e_ref, acc_ref, xw_ref, lambda a, b: a + b)

    @pl.when(i == NCHUNK - 1)
    def _():
        cs = jnp.sum(p_ref[...] * wsr_ref[...], axis=1, keepdims=True)
        t = jnp.maximum(cs + acc_ref[...] + b_ref[...], 0.0)
        out_ref[...] = _bn(t, g_ref[...], be_ref[...])


def _readout_kernel(p_ref, mem_ref, gp_ref, gn_ref, wd1_ref, bd1_ref,
                    g3_ref, be3_ref, wd2_ref, bd2_ref, wd3_ref, bd3_ref,
                    out_ref, h3_ref, sum_ref, max_ref):
    hcol = p_ref[:, 0:1]
    h3 = jnp.maximum(hcol * wd1_ref[...] + bd1_ref[...], 0.0)
    h3_ref[...] = _bn(h3, g3_ref[...], be3_ref[...])
    sum_ref[...] = jnp.zeros_like(sum_ref)
    max_ref[...] = jnp.full(max_ref.shape, -jnp.inf, dtype=jnp.float32)

    def body(i, carry):
        g = mem_ref[i]
        row = h3_ref[pl.ds(i, 1), :]
        sum_ref[pl.ds(g, 1), :] = sum_ref[pl.ds(g, 1), :] + row
        max_ref[pl.ds(g, 1), :] = jnp.maximum(max_ref[pl.ds(g, 1), :], row)
        return carry

    jax.lax.fori_loop(0, N, body, 0)

    sp = jnp.tanh(sum_ref[...])
    mp = jnp.tanh(jnp.maximum(max_ref[...], -1e9))
    logits = (jnp.sum(sp * wd2_ref[:, 0:3], axis=1, keepdims=True)
              + jnp.sum(mp * wd2_ref[:, 3:6], axis=1, keepdims=True)
              + bd2_ref[...])
    out_ref[...] = (logits * wd3_ref[:, 0:1]
                    + gp_ref[...] * wd3_ref[:, 1:2]
                    + gn_ref[...] * wd3_ref[:, 2:3]
                    + bd3_ref[...])


def _whole(shape):
    return pl.BlockSpec(shape, lambda i: tuple(0 for _ in shape))


_EDGE_SPEC = pl.BlockSpec((2, CHUNK), lambda i: (0, i),
                          memory_space=pltpu.SMEM)


@functools.partial(jax.jit, static_argnames=())
def kernel(x, edge_index, membership, g_pol, g_nonpol,
           W_self1, W_nei1, b1, gamma1, beta1,
           W_self2, W_nei2, b2, gamma2, beta2,
           Wd1, bd1, gamma3, beta3,
           Wd2, bd2, Wd3, bd3):
    f32 = jnp.float32
    nd = jax.ShapeDtypeStruct((N, D), f32)

    h1 = pl.pallas_call(
        _conv1_kernel,
        grid=(NCHUNK,),
        in_specs=[_whole((N, D)), _whole((D, D)), _whole((D, D)),
                  _whole((1, D)), _whole((1, D)), _whole((1, D)),
                  _EDGE_SPEC],
        out_specs=_whole((N, D)),
        out_shape=nd,
        scratch_shapes=[pltpu.VMEM((N, D), f32), pltpu.VMEM((N, D), f32)],
        compiler_params=_CP,
    )(x, W_nei1, W_self1, b1.reshape(1, D), gamma1.reshape(1, D),
      beta1.reshape(1, D), edge_index)

    pool = pl.pallas_call(
        _pool_kernel,
        grid=(NCHUNK,),
        in_specs=[_whole((N, D)), _EDGE_SPEC],
        out_specs=_whole((N, D)),
        out_shape=nd,
        scratch_shapes=[pltpu.VMEM((N, D), f32)],
        compiler_params=_CP,
    )

    p1 = pool(h1, edge_index)

    h2 = pl.pallas_call(
        _conv2_kernel,
        grid=(NCHUNK,),
        in_specs=[_whole((N, D)), _whole((1, D)), _whole((1, D)),
                  _whole((1, 1)), _whole((1, 1)), _whole((1, 1)),
                  _EDGE_SPEC],
        out_specs=_whole((N, D)),
        out_shape=nd,
        scratch_shapes=[pltpu.VMEM((N, D), f32), pltpu.VMEM((N, D), f32)],
        compiler_params=_CP,
    )(p1, W_self2.reshape(1, D), W_nei2.reshape(1, D), b2.reshape(1, 1),
      gamma2.reshape(1, 1), beta2.reshape(1, 1), edge_index)

    p2 = pool(h2, edge_index)

    out = pl.pallas_call(
        _readout_kernel,
        in_specs=[pl.BlockSpec((N, D), lambda: (0, 0)),
                  pl.BlockSpec(memory_space=pltpu.SMEM),
                  pl.BlockSpec((B, 1), lambda: (0, 0)),
                  pl.BlockSpec((B, 1), lambda: (0, 0)),
                  pl.BlockSpec((1, 3), lambda: (0, 0)),
                  pl.BlockSpec((1, 3), lambda: (0, 0)),
                  pl.BlockSpec((1, 3), lambda: (0, 0)),
                  pl.BlockSpec((1, 3), lambda: (0, 0)),
                  pl.BlockSpec((1, 6), lambda: (0, 0)),
                  pl.BlockSpec((1, 1), lambda: (0, 0)),
                  pl.BlockSpec((1, 3), lambda: (0, 0)),
                  pl.BlockSpec((1, 1), lambda: (0, 0))],
        out_specs=pl.BlockSpec((B, 1), lambda: (0, 0)),
        out_shape=jax.ShapeDtypeStruct((B, 1), f32),
        scratch_shapes=[pltpu.VMEM((N, 3), f32), pltpu.VMEM((B, 3), f32),
                        pltpu.VMEM((B, 3), f32)],
        compiler_params=pltpu.CompilerParams(
            vmem_limit_bytes=128 * 1024 * 1024),
    )(p2, membership, g_pol, g_nonpol,
      Wd1.reshape(1, 3), bd1.reshape(1, 3),
      gamma3.reshape(1, 3), beta3.reshape(1, 3),
      Wd2.reshape(1, 6), bd2.reshape(1, 1),
      Wd3.reshape(1, 3), bd3.reshape(1, 1))

    return out


# speedup vs baseline: 1.6100x; 1.6100x over previous
import functools

import jax
import jax.numpy as jnp
from jax.experimental import pallas as pl
from jax.experimental.pallas import tpu as pltpu

N = 10000
E = 320000
B = 100
D = 128
CHUNK = 16000
NCHUNK = E // CHUNK

_CP = pltpu.CompilerParams(
    dimension_semantics=("arbitrary",),
    vmem_limit_bytes=128 * 1024 * 1024,
)


_UNROLL = 4


def _scatter_loop(e_ref, acc_ref, src_ref, combine):
    def body(i, carry):
        e0 = i * _UNROLL
        for k in range(_UNROLL):
            e = e0 + k
            s = e_ref[0, e]
            d = e_ref[1, e]
            acc_ref[pl.ds(d, 1), :] = combine(
                acc_ref[pl.ds(d, 1), :], src_ref[pl.ds(s, 1), :]
            )
        return carry

    jax.lax.fori_loop(0, CHUNK // _UNROLL, body, 0)


def _bn(t, gamma, beta):
    mu = jnp.mean(t, axis=0, keepdims=True)
    var = jnp.var(t, axis=0, keepdims=True)
    return gamma * (t - mu) / jnp.sqrt(var + 1e-3) + beta


def _conv1_kernel(x_ref, wn_ref, ws_ref, b_ref, g_ref, be_ref, e_ref,
                  out_ref, xw_ref, acc_ref):
    i = pl.program_id(0)

    @pl.when(i == 0)
    def _():
        xw_ref[...] = jnp.dot(x_ref[...], wn_ref[...],
                              preferred_element_type=jnp.float32)
        acc_ref[...] = jnp.zeros_like(acc_ref)

    _scatter_loop(e_ref, acc_ref, xw_ref, lambda a, b: a + b)

    @pl.when(i == NCHUNK - 1)
    def _():
        t = jnp.dot(x_ref[...], ws_ref[...],
                    preferred_element_type=jnp.float32)
        t = jnp.maximum(t + acc_ref[...] + b_ref[...], 0.0)
        out_ref[...] = _bn(t, g_ref[...], be_ref[...])


def _pool_kernel(h_ref, e_ref, out_ref, acc_ref):
    i = pl.program_id(0)

    @pl.when(i == 0)
    def _():
        acc_ref[...] = jnp.full(acc_ref.shape, -jnp.inf, dtype=jnp.float32)

    _scatter_loop(e_ref, acc_ref, h_ref, jnp.maximum)

    @pl.when(i == NCHUNK - 1)
    def _():
        out_ref[...] = jnp.maximum(h_ref[...], acc_ref[...])


def _conv2_kernel(p_ref, wsr_ref, wnr_ref, b_ref, g_ref, be_ref, e_ref,
                  out_ref, xw_ref, acc_ref):
    i = pl.program_id(0)

    @pl.when(i == 0)
    def _():
        cn = jnp.sum(p_ref[...] * wnr_ref[...], axis=1, keepdims=True)
        xw_ref[...] = jnp.broadcast_to(cn, (N, D))
        acc_ref[...] = jnp.zeros_like(acc_ref)

    _scatter_loop(e_ref, acc_ref, xw_ref, lambda a, b: a + b)

    @pl.when(i == NCHUNK - 1)
    def _():
        cs = jnp.sum(p_ref[...] * wsr_ref[...], axis=1, keepdims=True)
        t = jnp.maximum(cs + acc_ref[...] + b_ref[...], 0.0)
        out_ref[...] = _bn(t, g_ref[...], be_ref[...])


def _readout_kernel(p_ref, mem_ref, gp_ref, gn_ref, wd1_ref, bd1_ref,
                    g3_ref, be3_ref, wd2_ref, bd2_ref, wd3_ref, bd3_ref,
                    out_ref, h3_ref, sum_ref, max_ref):
    hcol = p_ref[:, 0:1]
    h3 = jnp.maximum(hcol * wd1_ref[...] + bd1_ref[...], 0.0)
    h3_ref[...] = _bn(h3, g3_ref[...], be3_ref[...])
    sum_ref[...] = jnp.zeros_like(sum_ref)
    max_ref[...] = jnp.full(max_ref.shape, -jnp.inf, dtype=jnp.float32)

    def body(i, carry):
        g = mem_ref[i]
        row = h3_ref[pl.ds(i, 1), :]
        sum_ref[pl.ds(g, 1), :] = sum_ref[pl.ds(g, 1), :] + row
        max_ref[pl.ds(g, 1), :] = jnp.maximum(max_ref[pl.ds(g, 1), :], row)
        return carry

    jax.lax.fori_loop(0, N, body, 0)

    sp = jnp.tanh(sum_ref[...])
    mp = jnp.tanh(jnp.maximum(max_ref[...], -1e9))
    logits = (jnp.sum(sp * wd2_ref[:, 0:3], axis=1, keepdims=True)
              + jnp.sum(mp * wd2_ref[:, 3:6], axis=1, keepdims=True)
              + bd2_ref[...])
    out_ref[...] = (logits * wd3_ref[:, 0:1]
                    + gp_ref[...] * wd3_ref[:, 1:2]
                    + gn_ref[...] * wd3_ref[:, 2:3]
                    + bd3_ref[...])


def _whole(shape):
    return pl.BlockSpec(shape, lambda i: tuple(0 for _ in shape))


_EDGE_SPEC = pl.BlockSpec((2, CHUNK), lambda i: (0, i),
                          memory_space=pltpu.SMEM)


@functools.partial(jax.jit, static_argnames=())
def kernel(x, edge_index, membership, g_pol, g_nonpol,
           W_self1, W_nei1, b1, gamma1, beta1,
           W_self2, W_nei2, b2, gamma2, beta2,
           Wd1, bd1, gamma3, beta3,
           Wd2, bd2, Wd3, bd3):
    f32 = jnp.float32
    nd = jax.ShapeDtypeStruct((N, D), f32)

    h1 = pl.pallas_call(
        _conv1_kernel,
        grid=(NCHUNK,),
        in_specs=[_whole((N, D)), _whole((D, D)), _whole((D, D)),
                  _whole((1, D)), _whole((1, D)), _whole((1, D)),
                  _EDGE_SPEC],
        out_specs=_whole((N, D)),
        out_shape=nd,
        scratch_shapes=[pltpu.VMEM((N, D), f32), pltpu.VMEM((N, D), f32)],
        compiler_params=_CP,
    )(x, W_nei1, W_self1, b1.reshape(1, D), gamma1.reshape(1, D),
      beta1.reshape(1, D), edge_index)

    pool = pl.pallas_call(
        _pool_kernel,
        grid=(NCHUNK,),
        in_specs=[_whole((N, D)), _EDGE_SPEC],
        out_specs=_whole((N, D)),
        out_shape=nd,
        scratch_shapes=[pltpu.VMEM((N, D), f32)],
        compiler_params=_CP,
    )

    p1 = pool(h1, edge_index)

    h2 = pl.pallas_call(
        _conv2_kernel,
        grid=(NCHUNK,),
        in_specs=[_whole((N, D)), _whole((1, D)), _whole((1, D)),
                  _whole((1, 1)), _whole((1, 1)), _whole((1, 1)),
                  _EDGE_SPEC],
        out_specs=_whole((N, D)),
        out_shape=nd,
        scratch_shapes=[pltpu.VMEM((N, D), f32), pltpu.VMEM((N, D), f32)],
        compiler_params=_CP,
    )(p1, W_self2.reshape(1, D), W_nei2.reshape(1, D), b2.reshape(1, 1),
      gamma2.reshape(1, 1), beta2.reshape(1, 1), edge_index)

    p2 = pool(h2, edge_index)

    out = pl.pallas_call(
        _readout_kernel,
        in_specs=[pl.BlockSpec((N, D), lambda: (0, 0)),
                  pl.BlockSpec(memory_space=pltpu.SMEM),
                  pl.BlockSpec((B, 1), lambda: (0, 0)),
                  pl.BlockSpec((B, 1), lambda: (0, 0)),
                  pl.BlockSpec((1, 3), lambda: (0, 0)),
                  pl.BlockSpec((1, 3), lambda: (0, 0)),
                  pl.BlockSpec((1, 3), lambda: (0, 0)),
                  pl.BlockSpec((1, 3), lambda: (0, 0)),
                  pl.BlockSpec((1, 6), lambda: (0, 0)),
                  pl.BlockSpec((1, 1), lambda: (0, 0)),
                  pl.BlockSpec((1, 3), lambda: (0, 0)),
                  pl.BlockSpec((1, 1), lambda: (0, 0))],
        out_specs=pl.BlockSpec((B, 1), lambda: (0, 0)),
        out_shape=jax.ShapeDtypeStruct((B, 1), f32),
        scratch_shapes=[pltpu.VMEM((N, 3), f32), pltpu.VMEM((B, 3), f32),
                        pltpu.VMEM((B, 3), f32)],
        compiler_params=pltpu.CompilerParams(
            vmem_limit_bytes=128 * 1024 * 1024),
    )(p2, membership, g_pol, g_nonpol,
      Wd1.reshape(1, 3), bd1.reshape(1, 3),
      gamma3.reshape(1, 3), beta3.reshape(1, 3),
      Wd2.reshape(1, 6), bd2.reshape(1, 1),
      Wd3.reshape(1, 3), bd3.reshape(1, 1))

    return out
